# baseline (speedup 1.0000x reference)
import functools

import jax
import jax.numpy as jnp
from jax import lax
from jax.experimental import pallas as pl
from jax.experimental.pallas import tpu as pltpu
from jax.experimental.pallas import tpu_sc as plsc

NUM_CORES = 2
NUM_SUBCORES = 16
NUM_WORKERS = NUM_CORES * NUM_SUBCORES

BATCH = 16384
HIST_LEN = 20
FEATURES = 64
B = BATCH * HIST_LEN
BLK = 128
NBLK = B // BLK
BLK_PER_W = NBLK // NUM_WORKERS
IDX_PER_W = B // NUM_WORKERS
CBLK = BATCH // BLK


def _embed_kernel(table_hbm, idx_hbm, out_hbm,
                  idx_v, rows_v, stage_v, gsem0, gsem1, osem0, osem1):
    wid = lax.axis_index("s") * NUM_CORES + lax.axis_index("c")
    gsems = (gsem0, gsem1)
    osems = (osem0, osem1)

    pltpu.async_copy(
        idx_hbm.at[pl.ds(wid * IDX_PER_W, IDX_PER_W)], idx_v, gsem0
    ).wait()

    def gather_desc(i, buf):
        return pltpu.make_async_copy(
            table_hbm.at[idx_v.at[pl.ds(i * BLK, BLK)]],
            rows_v.at[buf],
            gsems[buf],
        )

    def out_desc(i, buf):
        blk = wid * BLK_PER_W + i
        h = blk // CBLK
        c = blk % CBLK
        return pltpu.make_async_copy(
            stage_v.at[buf],
            out_hbm.at[h, :, c, :, :],
            osems[buf],
        )

    svec = lax.iota(jnp.int32, 16)
    rowv = [svec + 16 * g for g in range(BLK // 16)]
    fvec = [jnp.full((16,), f, jnp.int32) for f in range(FEATURES)]

    def extract(buf):
        for f in range(FEATURES):
            tf, sf = f // 8, f % 8
            for g in range(BLK // 16):
                vals = plsc.load_gather(rows_v.at[buf], [rowv[g], fvec[f]])
                stage_v[buf, tf, sf, pl.ds(16 * g, 16)] = vals

    def step(i, buf):
        gather_desc(i, buf).wait()

        @pl.when(i >= 2)
        def _():
            out_desc(i - 2, buf).wait()
        extract(buf)
        out_desc(i, buf).start()

        @pl.when(i + 2 < BLK_PER_W)
        def _():
            gather_desc(i + 2, buf).start()

    gather_desc(0, 0).start()
    gather_desc(1, 1).start()

    @pl.loop(0, BLK_PER_W, step=2)
    def _(i):
        step(i, 0)
        step(i + 1, 1)

    out_desc(BLK_PER_W - 2, 0).wait()
    out_desc(BLK_PER_W - 1, 1).wait()


@jax.jit
def kernel(inputs, embedding):
    idx_flat = inputs.T.reshape(-1).astype(jnp.int32)
    mesh = plsc.VectorSubcoreMesh(
        core_axis_name="c", subcore_axis_name="s",
        num_cores=NUM_CORES, num_subcores=NUM_SUBCORES,
    )
    run = pl.kernel(
        _embed_kernel,
        out_type=jax.ShapeDtypeStruct(
            (HIST_LEN, FEATURES // 8, CBLK, 8, BLK), jnp.float32
        ),
        mesh=mesh,
        scratch_types=[
            pltpu.VMEM((IDX_PER_W,), jnp.int32),
            pltpu.VMEM((2, BLK, FEATURES), jnp.float32),
            pltpu.VMEM((2, FEATURES // 8, 8, BLK), jnp.float32),
            pltpu.SemaphoreType.DMA,
            pltpu.SemaphoreType.DMA,
            pltpu.SemaphoreType.DMA,
            pltpu.SemaphoreType.DMA,
        ],
        compiler_params=pltpu.CompilerParams(
            use_tc_tiling_on_sc=False, needs_layout_passes=False
        ),
    )
    out5 = run(embedding, idx_flat)
    return out5.transpose(2, 4, 0, 1, 3).reshape(BATCH, HIST_LEN, FEATURES)


# speedup vs baseline: 1.3871x; 1.3871x over previous
import functools

import jax
import jax.numpy as jnp
from jax import lax
from jax.experimental import pallas as pl
from jax.experimental.pallas import tpu as pltpu
from jax.experimental.pallas import tpu_sc as plsc

NUM_CORES = 2
NUM_SUBCORES = 16
NUM_WORKERS = NUM_CORES * NUM_SUBCORES

BATCH = 16384
HIST_LEN = 20
FEATURES = 64
B = BATCH * HIST_LEN
B_PER_W = B // NUM_WORKERS
CHUNK = 512
NCHUNK = B_PER_W // CHUNK


def _embed_kernel(table_hbm, idx_hbm, out_hbm, idx_v, rows_v, gsem):
    wid = lax.axis_index("s") * NUM_CORES + lax.axis_index("c")
    base = wid * B_PER_W

    pltpu.sync_copy(idx_hbm.at[pl.ds(base, B_PER_W)], idx_v)

    def start_gather(i, buf):
        pltpu.async_copy(
            table_hbm.at[idx_v.at[pl.ds(i * CHUNK, CHUNK)]],
            rows_v.at[buf],
            gsem,
        )

    def finish_and_store(i, buf):
        pltpu.make_async_copy(
            table_hbm.at[idx_v.at[pl.ds(i * CHUNK, CHUNK)]],
            rows_v.at[buf],
            gsem,
        ).wait()
        pltpu.sync_copy(rows_v.at[buf], out_hbm.at[pl.ds(base + i * CHUNK, CHUNK)])

    start_gather(0, 0)

    @pl.loop(0, NCHUNK, step=2)
    def _(i):
        start_gather(i + 1, 1)
        finish_and_store(i, 0)
        @pl.when(i + 2 < NCHUNK)
        def _():
            start_gather(i + 2, 0)
        finish_and_store(i + 1, 1)


@jax.jit
def kernel(inputs, embedding):
    idx_flat = inputs.T.reshape(-1).astype(jnp.int32)
    mesh = plsc.VectorSubcoreMesh(
        core_axis_name="c", subcore_axis_name="s",
        num_cores=NUM_CORES, num_subcores=NUM_SUBCORES,
    )
    run = pl.kernel(
        _embed_kernel,
        out_type=jax.ShapeDtypeStruct((B, FEATURES), jnp.float32),
        mesh=mesh,
        scratch_types=[
            pltpu.VMEM((B_PER_W,), jnp.int32),
            pltpu.VMEM((2, CHUNK, FEATURES), jnp.float32),
            pltpu.SemaphoreType.DMA,
        ],
        compiler_params=pltpu.CompilerParams(use_tc_tiling_on_sc=False),
    )
    out = run(embedding, idx_flat)
    return out.reshape(HIST_LEN, BATCH, FEATURES).transpose(1, 0, 2)


# speedup vs baseline: 1.3877x; 1.0004x over previous
import jax
import jax.numpy as jnp
from jax import lax
from jax.experimental import pallas as pl
from jax.experimental.pallas import tpu as pltpu
from jax.experimental.pallas import tpu_sc as plsc

NUM_CORES = 2
NUM_SUBCORES = 16
NUM_WORKERS = NUM_CORES * NUM_SUBCORES

BATCH = 16384
HIST_LEN = 20
FEATURES = 64
B = BATCH * HIST_LEN
B_PER_W = B // NUM_WORKERS
CHUNK = 512
NCHUNK = B_PER_W // CHUNK


def _embed_kernel(table_hbm, idx_hbm, out_hbm, idx_v, rows_v, sem0, sem1):
    sems = (sem0, sem1)
    wid = lax.axis_index("s") * NUM_CORES + lax.axis_index("c")
    base = wid * B_PER_W

    pltpu.sync_copy(idx_hbm.at[pl.ds(base, B_PER_W)], idx_v)

    def start_gather(i, buf):
        pltpu.async_copy(
            table_hbm.at[idx_v.at[pl.ds(i * CHUNK, CHUNK)]],
            rows_v.at[buf],
            sems[buf],
        )

    def finish_and_store(i, buf):
        pltpu.make_async_copy(
            table_hbm.at[idx_v.at[pl.ds(i * CHUNK, CHUNK)]],
            rows_v.at[buf],
            sems[buf],
        ).wait()
        pltpu.sync_copy(rows_v.at[buf], out_hbm.at[pl.ds(base + i * CHUNK, CHUNK)])

    start_gather(0, 0)

    @pl.loop(0, NCHUNK, step=2)
    def _(i):
        start_gather(i + 1, 1)
        finish_and_store(i, 0)
        @pl.when(i + 2 < NCHUNK)
        def _():
            start_gather(i + 2, 0)
        finish_and_store(i + 1, 1)


@jax.jit
def kernel(inputs, embedding):
    idx_flat = inputs.T.reshape(-1).astype(jnp.int32)
    mesh = plsc.VectorSubcoreMesh(
        core_axis_name="c", subcore_axis_name="s",
        num_cores=NUM_CORES, num_subcores=NUM_SUBCORES,
    )
    run = pl.kernel(
        _embed_kernel,
        out_type=jax.ShapeDtypeStruct((B, FEATURES), jnp.float32),
        mesh=mesh,
        scratch_types=[
            pltpu.VMEM((B_PER_W,), jnp.int32),
            pltpu.VMEM((2, CHUNK, FEATURES), jnp.float32),
            pltpu.SemaphoreType.DMA,
            pltpu.SemaphoreType.DMA,
        ],
        compiler_params=pltpu.CompilerParams(use_tc_tiling_on_sc=False),
    )
    out = run(embedding, idx_flat)
    return out.reshape(HIST_LEN, BATCH, FEATURES).transpose(1, 0, 2)
